# baseline (speedup 1.0000x reference)
import functools

import jax
import jax.numpy as jnp
from jax import lax
from jax.experimental import pallas as pl
from jax.experimental.pallas import tpu as pltpu
from jax.experimental.pallas import tpu_sc as plsc


@functools.lru_cache(maxsize=None)
def _build(B, D, V):
    info = plsc.get_sparse_core_info()
    NC, NS, L = info.num_cores, info.num_subcores, info.num_lanes
    NW = NC * NS
    assert B % (NW * L) == 0 and D % L == 0
    PER_W = B // NW
    MYV = PER_W // L
    ROWS = 32
    NBUF = 3
    NCH = PER_W // ROWS
    VPC = ROWS // L
    K = 16
    NB = B // (K * L)

    mesh = plsc.VectorSubcoreMesh(core_axis_name="c", subcore_axis_name="s")

    @functools.partial(
        pl.kernel,
        mesh=mesh,
        compiler_params=pltpu.CompilerParams(needs_layout_passes=False),
        out_type=jax.ShapeDtypeStruct((B, D), jnp.float32),
        scratch_types=[
            pltpu.VMEM((V,), jnp.int32),
            pltpu.VMEM((B,), jnp.int32),
            pltpu.VMEM((NCH, ROWS), jnp.int32),
            pltpu.VMEM((NBUF, ROWS, D), jnp.float32),
            pltpu.SemaphoreType.DMA,
        ],
    )
    def k(idx_hbm, val_hbm, out_hbm, t_ref, idx_v, j_ref, rows_v, sem_r):
        wid = lax.axis_index("s") * NC + lax.axis_index("c")
        base = wid * PER_W
        lanes = lax.iota(jnp.int32, L)
        pltpu.sync_copy(idx_hbm, idx_v)

        def a_body(v, carry):
            off = pl.multiple_of(v * (K * L), K * L)
            for kk in range(K):
                vec = idx_v[pl.ds(off + kk * L, L)]
                pos = off + kk * L + lanes
                _, last = plsc.scan_count(vec)
                plsc.store_scatter(t_ref, [vec], pos, mask=last)
            return carry

        lax.fori_loop(0, NB, a_body, jnp.int32(0))

        for u in range(MYV):
            vec = idx_v[pl.ds(base + u * L, L)]
            j = plsc.load_gather(t_ref, [vec])
            j_ref[u // VPC, pl.ds((u % VPC) * L, L)] = j

        def row_dma(c):
            return pltpu.async_copy(
                val_hbm.at[j_ref.at[c]], rows_v.at[c % NBUF], sem_r)

        handles = {}
        for c in range(min(NBUF - 1, NCH)):
            handles[c] = row_dma(c)
        for c in range(NCH):
            if c + NBUF - 1 < NCH:
                handles[c + NBUF - 1] = row_dma(c + NBUF - 1)
            handles[c].wait()
            pltpu.sync_copy(rows_v.at[c % NBUF],
                            out_hbm.at[pl.ds(base + c * ROWS, ROWS)])

    return k


def kernel(memory, node_idxs, values):
    B, D = values.shape
    V = memory.shape[0]
    return _build(B, D, V)(node_idxs, values)


# speedup vs baseline: 1.3879x; 1.3879x over previous
import functools

import jax
import jax.numpy as jnp
from jax import lax
from jax.experimental import pallas as pl
from jax.experimental.pallas import tpu as pltpu
from jax.experimental.pallas import tpu_sc as plsc


@functools.lru_cache(maxsize=None)
def _build(B, D, V):
    info = plsc.get_sparse_core_info()
    NC, NS, L = info.num_cores, info.num_subcores, info.num_lanes
    NW = NC * NS
    assert B % (NW * L) == 0 and D % L == 0
    PER_W = B // NW
    MYV = PER_W // L
    ROWS = 32
    NBUF = 3
    NCH = PER_W // ROWS
    VPC = ROWS // L
    K = 16
    NB = B // (K * L)

    mesh = plsc.VectorSubcoreMesh(core_axis_name="c", subcore_axis_name="s")

    @functools.partial(
        pl.kernel,
        mesh=mesh,
        compiler_params=pltpu.CompilerParams(needs_layout_passes=False),
        out_type=jax.ShapeDtypeStruct((B, D), jnp.float32),
        scratch_types=[
            pltpu.VMEM((V,), jnp.int32),
            pltpu.VMEM((B,), jnp.int32),
            pltpu.VMEM((NCH, ROWS), jnp.int32),
            pltpu.VMEM((NBUF, ROWS, D), jnp.float32),
            pltpu.SemaphoreType.DMA,
        ],
    )
    def k(idx_hbm, val_hbm, out_hbm, t_ref, idx_v, j_ref, rows_v, sem_r):
        wid = lax.axis_index("s") * NC + lax.axis_index("c")
        base = wid * PER_W
        lanes = lax.iota(jnp.int32, L)
        pltpu.sync_copy(idx_hbm, idx_v)

        def batch(v, fix):
            off = pl.multiple_of(v * (K * L), K * L)
            vecs = [idx_v[pl.ds(off + k * L, L)] for k in range(K)]
            poss = [off + k * L + lanes for k in range(K)]
            if not fix:
                for kk in range(K):
                    plsc.store_scatter(t_ref, [vecs[kk]], poss[kk])
            gs = [plsc.load_gather(t_ref, [vecs[kk]]) for kk in range(K)]
            ms = [poss[kk] > gs[kk] for kk in range(K)]
            if fix:
                for kk in range(K):
                    plsc.store_scatter(t_ref, [vecs[kk]], poss[kk],
                                       mask=ms[kk])
            while len(ms) > 1:
                ms = [a | b for a, b in zip(ms[::2], ms[1::2])]
            return ms[0]

        acc = lax.fori_loop(
            0, NB, lambda v, a: a | batch(v, fix=False), lanes < 0)

        def fix_pass(_):
            return jnp.any(lax.fori_loop(
                0, NB, lambda v, a: a | batch(v, fix=True), lanes < 0))

        lax.while_loop(lambda d: d, fix_pass, jnp.any(acc))

        for u in range(MYV):
            vec = idx_v[pl.ds(base + u * L, L)]
            j = plsc.load_gather(t_ref, [vec])
            j_ref[u // VPC, pl.ds((u % VPC) * L, L)] = j

        def row_dma(c):
            return pltpu.async_copy(
                val_hbm.at[j_ref.at[c]], rows_v.at[c % NBUF], sem_r)

        handles = {}
        for c in range(min(NBUF - 1, NCH)):
            handles[c] = row_dma(c)
        for c in range(NCH):
            if c + NBUF - 1 < NCH:
                handles[c + NBUF - 1] = row_dma(c + NBUF - 1)
            handles[c].wait()
            pltpu.sync_copy(rows_v.at[c % NBUF],
                            out_hbm.at[pl.ds(base + c * ROWS, ROWS)])

    return k


def kernel(memory, node_idxs, values):
    B, D = values.shape
    V = memory.shape[0]
    return _build(B, D, V)(node_idxs, values)


# speedup vs baseline: 1.3933x; 1.0039x over previous
import functools

import jax
import jax.numpy as jnp
from jax import lax
from jax.experimental import pallas as pl
from jax.experimental.pallas import tpu as pltpu
from jax.experimental.pallas import tpu_sc as plsc


@functools.lru_cache(maxsize=None)
def _build(B, D, V):
    info = plsc.get_sparse_core_info()
    NC, NS, L = info.num_cores, info.num_subcores, info.num_lanes
    NW = NC * NS
    assert B % (NW * L) == 0 and D % L == 0
    PER_W = B // NW
    MYV = PER_W // L
    ROWS = 32
    NBUF = 3
    NCH = PER_W // ROWS
    VPC = ROWS // L
    K = 8
    NB = B // (K * L)

    mesh = plsc.VectorSubcoreMesh(core_axis_name="c", subcore_axis_name="s")

    @functools.partial(
        pl.kernel,
        mesh=mesh,
        compiler_params=pltpu.CompilerParams(needs_layout_passes=False),
        out_type=jax.ShapeDtypeStruct((B, D), jnp.float32),
        scratch_types=[
            pltpu.VMEM((V,), jnp.int32),
            pltpu.VMEM((B,), jnp.int32),
            pltpu.VMEM((NCH, ROWS), jnp.int32),
            pltpu.VMEM((NBUF, ROWS, D), jnp.float32),
            pltpu.SemaphoreType.DMA,
        ],
    )
    def k(idx_hbm, val_hbm, out_hbm, t_ref, idx_v, j_ref, rows_v, sem_r):
        wid = lax.axis_index("s") * NC + lax.axis_index("c")
        base = wid * PER_W
        lanes = lax.iota(jnp.int32, L)
        pltpu.sync_copy(idx_hbm, idx_v)

        def batch(v, fix):
            off = pl.multiple_of(v * (K * L), K * L)
            vecs = [idx_v[pl.ds(off + k * L, L)] for k in range(K)]
            poss = [off + k * L + lanes for k in range(K)]
            if not fix:
                for kk in range(K):
                    plsc.store_scatter(t_ref, [vecs[kk]], poss[kk])
            gs = [plsc.load_gather(t_ref, [vecs[kk]]) for kk in range(K)]
            ms = [poss[kk] > gs[kk] for kk in range(K)]
            if fix:
                for kk in range(K):
                    plsc.store_scatter(t_ref, [vecs[kk]], poss[kk],
                                       mask=ms[kk])
            while len(ms) > 1:
                ms = [a | b for a, b in zip(ms[::2], ms[1::2])]
            return ms[0]

        acc = lax.fori_loop(
            0, NB, lambda v, a: a | batch(v, fix=False), lanes < 0)

        def fix_pass(_):
            return jnp.any(lax.fori_loop(
                0, NB, lambda v, a: a | batch(v, fix=True), lanes < 0))

        lax.while_loop(lambda d: d, fix_pass, jnp.any(acc))

        for u in range(MYV):
            vec = idx_v[pl.ds(base + u * L, L)]
            j = plsc.load_gather(t_ref, [vec])
            j_ref[u // VPC, pl.ds((u % VPC) * L, L)] = j

        def row_dma(c):
            return pltpu.async_copy(
                val_hbm.at[j_ref.at[c]], rows_v.at[c % NBUF], sem_r)

        handles = {}
        for c in range(min(NBUF - 1, NCH)):
            handles[c] = row_dma(c)
        for c in range(NCH):
            if c + NBUF - 1 < NCH:
                handles[c + NBUF - 1] = row_dma(c + NBUF - 1)
            handles[c].wait()
            pltpu.sync_copy(rows_v.at[c % NBUF],
                            out_hbm.at[pl.ds(base + c * ROWS, ROWS)])

    return k


def kernel(memory, node_idxs, values):
    B, D = values.shape
    V = memory.shape[0]
    return _build(B, D, V)(node_idxs, values)
